# baseline (speedup 1.0000x reference)
import functools

import jax
import jax.numpy as jnp
from jax import lax
from jax.experimental import pallas as pl
from jax.experimental.pallas import tpu as pltpu
from jax.experimental.pallas import tpu_sc as plsc

B = 1024
TLEN = 2048
GLEN = 200
D = 64
NTOK = 4
NT = B * TLEN
NG = B * GLEN

NC = 2
NS = 16
NW = NC * NS

T_PER_W = NT // NW
G_PER_W = NG // NW
BLK = 256
GPB = BLK // 128
NBUF = 4
T_ROUNDS = T_PER_W // (BLK * NBUF)
G_ROUNDS = G_PER_W // (BLK * NBUF)
G_LEFT = G_PER_W // BLK - G_ROUNDS * NBUF


def _table_body(nuc_ref, pos_ref, out_ref):
    for t in range(NTOK):
        out_ref[pl.ds(t * TLEN, TLEN), :] = pos_ref[...] + nuc_ref[pl.ds(t, 1), :]


_build_table = pl.pallas_call(
    _table_body,
    out_shape=jax.ShapeDtypeStruct((NTOK * TLEN, D), jnp.float32),
)


def _combine_body(tok_ref, pos_ref, out_ref):
    out_ref[...] = tok_ref[...] * TLEN + pos_ref[...]


_combine_target = pl.pallas_call(
    _combine_body,
    grid=(8,),
    in_specs=[
        pl.BlockSpec((B // 8, TLEN), lambda i: (i, 0)),
        pl.BlockSpec((B // 8, TLEN), lambda i: (i, 0)),
    ],
    out_specs=pl.BlockSpec((B // 8, TLEN), lambda i: (i, 0)),
    out_shape=jax.ShapeDtypeStruct((B, TLEN), jnp.int32),
)

_combine_guide = pl.pallas_call(
    _combine_body,
    out_shape=jax.ShapeDtypeStruct((B, GLEN), jnp.int32),
)


def _retile_t_body(in_ref, out_ref):
    out_ref[0] = jnp.swapaxes(in_ref[...], 0, 1)


_retile_target = pl.pallas_call(
    _retile_t_body,
    grid=(B,),
    in_specs=[pl.BlockSpec((TLEN, D), lambda b: (b, 0))],
    out_specs=pl.BlockSpec((1, D, TLEN), lambda b: (b, 0, 0)),
    out_shape=jax.ShapeDtypeStruct((B, D, TLEN), jnp.float32),
)


def _retile_g_body(in_ref, out_ref):
    for k in range(8):
        out_ref[k] = in_ref[:, k, :].T


_retile_guide = pl.pallas_call(
    _retile_g_body,
    grid=(GLEN // 8,),
    in_specs=[pl.BlockSpec((B, 8, D), lambda s: (0, s, 0))],
    out_specs=pl.BlockSpec((8, D, B), lambda s: (s, 0, 0)),
    out_shape=jax.ShapeDtypeStruct((GLEN, D, B), jnp.float32),
)


def _run_chunk(table_hbm, idx_hbm, out_hbm, idx_v, rows_v, sem_i, sem_g, sem_w,
               start, n_rounds, leftover):
    def stage(j, b):
        pltpu.async_copy(idx_hbm.at[pl.ds(start + j * BLK, BLK)],
                         idx_v.at[b], sem_i[b])

    def fire_gathers(b):
        for q in range(GPB):
            pltpu.async_copy(table_hbm.at[idx_v.at[b, pl.ds(q * 128, 128)]],
                             rows_v.at[b, pl.ds(q * 128, 128)], sem_g[b])

    def fire_write(j, b):
        pltpu.async_copy(rows_v.at[b], out_hbm.at[pl.ds(start + j * BLK, BLK)],
                         sem_w[b])

    def drain_i(b):
        pltpu.make_async_copy(idx_hbm.at[pl.ds(0, BLK)], idx_v.at[b],
                              sem_i[b]).wait()

    def drain_g(b):
        pltpu.make_async_copy(out_hbm.at[pl.ds(0, BLK)], rows_v.at[b],
                              sem_g[b]).wait()

    def drain_w(b):
        pltpu.make_async_copy(rows_v.at[b], out_hbm.at[pl.ds(0, BLK)],
                              sem_w[b]).wait()

    for b in range(NBUF):
        stage(b, b)
    for b in range(NBUF):
        drain_i(b)
        fire_gathers(b)

    def round_body(g, carry):
        for b in range(NBUF):
            drain_g(b)
            fire_write(NBUF * (g - 1) + b, b)
        for b in range(NBUF):
            stage(NBUF * g + b, b)
        for b in range(NBUF):
            drain_i(b)
            drain_w(b)
            fire_gathers(b)
        return carry

    if n_rounds > 1:
        lax.fori_loop(1, n_rounds, round_body, 0)

    for b in range(NBUF):
        drain_g(b)
        fire_write(NBUF * (n_rounds - 1) + b, b)
    for b in range(NBUF):
        drain_w(b)

    for k in range(leftover):
        j = NBUF * n_rounds + k
        stage(j, 0)
        drain_i(0)
        fire_gathers(0)
        drain_g(0)
        fire_write(j, 0)
        drain_w(0)


def _sc_body(table_hbm, tidx_hbm, gidx_hbm, out_t, out_g, idx_v, rows_v, *sems):
    sem_i, sem_g, sem_w = sems[0:NBUF], sems[NBUF:2 * NBUF], sems[2 * NBUF:]
    wid = lax.axis_index("s") * NC + lax.axis_index("c")
    _run_chunk(table_hbm, tidx_hbm, out_t, idx_v, rows_v, sem_i, sem_g, sem_w,
               wid * T_PER_W, T_ROUNDS, 0)
    _run_chunk(table_hbm, gidx_hbm, out_g, idx_v, rows_v, sem_i, sem_g, sem_w,
               wid * G_PER_W, G_ROUNDS, G_LEFT)


_sc_gather = functools.partial(
    pl.kernel,
    out_type=[
        jax.ShapeDtypeStruct((NT, D), jnp.float32),
        jax.ShapeDtypeStruct((NG, D), jnp.float32),
    ],
    mesh=plsc.VectorSubcoreMesh(core_axis_name="c", subcore_axis_name="s"),
    compiler_params=pltpu.CompilerParams(use_tc_tiling_on_sc=False),
    scratch_types=(
        [pltpu.VMEM((NBUF, BLK), jnp.int32),
         pltpu.VMEM((NBUF, BLK, D), jnp.float32)]
        + [pltpu.SemaphoreType.DMA] * (3 * NBUF)
    ),
)(_sc_body)


def kernel(target_tokens, target_positions, guide_tokens, guide_positions,
           nuc_table, pos_table):
    tt = target_tokens.astype(jnp.int32)
    tp = target_positions.astype(jnp.int32)
    gt = guide_tokens.astype(jnp.int32)
    gp = guide_positions.astype(jnp.int32)

    table = _build_table(nuc_table, pos_table)
    tidx = _combine_target(tt, tp).reshape(NT)
    gidx = _combine_guide(gt, gp).reshape(NG)

    out_t, out_g = _sc_gather(table, tidx, gidx)

    out_t3 = _retile_target(out_t)
    out_g3 = _retile_guide(out_g.reshape(B, GLEN, D))
    res_t = jnp.transpose(out_t3, (0, 2, 1))
    res_g = jnp.transpose(out_g3, (2, 0, 1))
    return res_t, res_g


# speedup vs baseline: 1.3864x; 1.3864x over previous
import functools

import jax
import jax.numpy as jnp
from jax import lax
from jax.experimental import pallas as pl
from jax.experimental.pallas import tpu as pltpu
from jax.experimental.pallas import tpu_sc as plsc

B = 1024
TLEN = 2048
GLEN = 200
D = 64
NTOK = 4
NT = B * TLEN
NG = B * GLEN

NC = 2
NS = 16
NW = NC * NS

T_PER_W = NT // NW
G_PER_W = NG // NW
BLK = 256
GPB = BLK // 128
NBUF = 4
T_ROUNDS = T_PER_W // (BLK * NBUF)
G_ROUNDS = G_PER_W // (BLK * NBUF)
G_LEFT = G_PER_W // BLK - G_ROUNDS * NBUF


def _table_body(nuc_ref, pos_ref, out_ref):
    for t in range(NTOK):
        out_ref[pl.ds(t * TLEN, TLEN), :] = pos_ref[...] + nuc_ref[pl.ds(t, 1), :]


_build_table = pl.pallas_call(
    _table_body,
    out_shape=jax.ShapeDtypeStruct((NTOK * TLEN, D), jnp.float32),
)


def _combine_body(tok_ref, pos_ref, out_ref):
    out_ref[...] = tok_ref[...] * TLEN + pos_ref[...]


_combine_target = pl.pallas_call(
    _combine_body,
    grid=(8,),
    in_specs=[
        pl.BlockSpec((B // 8, TLEN), lambda i: (i, 0)),
        pl.BlockSpec((B // 8, TLEN), lambda i: (i, 0)),
    ],
    out_specs=pl.BlockSpec((B // 8, TLEN), lambda i: (i, 0)),
    out_shape=jax.ShapeDtypeStruct((B, TLEN), jnp.int32),
)

_combine_guide = pl.pallas_call(
    _combine_body,
    out_shape=jax.ShapeDtypeStruct((B, GLEN), jnp.int32),
)


def _retile_g_body(in_ref, out_ref):
    for k in range(8):
        out_ref[k] = in_ref[:, k, :].T


_retile_guide = pl.pallas_call(
    _retile_g_body,
    grid=(GLEN // 8,),
    in_specs=[pl.BlockSpec((B, 8, D), lambda s: (0, s, 0))],
    out_specs=pl.BlockSpec((8, D, B), lambda s: (s, 0, 0)),
    out_shape=jax.ShapeDtypeStruct((GLEN, D, B), jnp.float32),
)


def _run_chunk(table_hbm, idx_hbm, out_hbm, idx_v, rows_v, sem_i, sem_g, sem_w,
               start, n_rounds, leftover):
    def stage(j, b):
        pltpu.async_copy(idx_hbm.at[pl.ds(start + j * BLK, BLK)],
                         idx_v.at[b], sem_i[b])

    def fire_gathers(b):
        for q in range(GPB):
            pltpu.async_copy(table_hbm.at[idx_v.at[b, pl.ds(q * 128, 128)]],
                             rows_v.at[b, pl.ds(q * 128, 128)], sem_g[b])

    def fire_write(j, b):
        pltpu.async_copy(rows_v.at[b], out_hbm.at[pl.ds(start + j * BLK, BLK)],
                         sem_w[b])

    def drain_i(b):
        pltpu.make_async_copy(idx_hbm.at[pl.ds(0, BLK)], idx_v.at[b],
                              sem_i[b]).wait()

    def drain_g(b):
        pltpu.make_async_copy(out_hbm.at[pl.ds(0, BLK)], rows_v.at[b],
                              sem_g[b]).wait()

    def drain_w(b):
        pltpu.make_async_copy(rows_v.at[b], out_hbm.at[pl.ds(0, BLK)],
                              sem_w[b]).wait()

    for b in range(NBUF):
        stage(b, b)
    for b in range(NBUF):
        drain_i(b)
        fire_gathers(b)

    def round_body(g, carry):
        for b in range(NBUF):
            drain_g(b)
            fire_write(NBUF * (g - 1) + b, b)
        for b in range(NBUF):
            stage(NBUF * g + b, b)
        for b in range(NBUF):
            drain_i(b)
            drain_w(b)
            fire_gathers(b)
        return carry

    if n_rounds > 1:
        lax.fori_loop(1, n_rounds, round_body, 0)

    for b in range(NBUF):
        drain_g(b)
        fire_write(NBUF * (n_rounds - 1) + b, b)
    for b in range(NBUF):
        drain_w(b)

    for k in range(leftover):
        j = NBUF * n_rounds + k
        stage(j, 0)
        drain_i(0)
        fire_gathers(0)
        drain_g(0)
        fire_write(j, 0)
        drain_w(0)


def _sc_body(table_hbm, tidx_hbm, gidx_hbm, out_t, out_g, idx_v, rows_v, *sems):
    sem_i, sem_g, sem_w = sems[0:NBUF], sems[NBUF:2 * NBUF], sems[2 * NBUF:]
    wid = lax.axis_index("s") * NC + lax.axis_index("c")
    _run_chunk(table_hbm, tidx_hbm, out_t, idx_v, rows_v, sem_i, sem_g, sem_w,
               wid * T_PER_W, T_ROUNDS, 0)
    _run_chunk(table_hbm, gidx_hbm, out_g, idx_v, rows_v, sem_i, sem_g, sem_w,
               wid * G_PER_W, G_ROUNDS, G_LEFT)


_sc_gather = functools.partial(
    pl.kernel,
    out_type=[
        jax.ShapeDtypeStruct((NT, D), jnp.float32),
        jax.ShapeDtypeStruct((NG, D), jnp.float32),
    ],
    mesh=plsc.VectorSubcoreMesh(core_axis_name="c", subcore_axis_name="s"),
    compiler_params=pltpu.CompilerParams(use_tc_tiling_on_sc=False),
    scratch_types=(
        [pltpu.VMEM((NBUF, BLK), jnp.int32),
         pltpu.VMEM((NBUF, BLK, D), jnp.float32)]
        + [pltpu.SemaphoreType.DMA] * (3 * NBUF)
    ),
)(_sc_body)


def kernel(target_tokens, target_positions, guide_tokens, guide_positions,
           nuc_table, pos_table):
    tt = target_tokens.astype(jnp.int32)
    tp = target_positions.astype(jnp.int32)
    gt = guide_tokens.astype(jnp.int32)
    gp = guide_positions.astype(jnp.int32)

    table = _build_table(nuc_table, pos_table)
    tidx = _combine_target(tt, tp).reshape(NT)
    gidx = _combine_guide(gt, gp).reshape(NG)

    out_t, out_g = _sc_gather(table, tidx, gidx)

    out_g3 = _retile_guide(out_g.reshape(B, GLEN, D))
    res_g = jnp.transpose(out_g3, (2, 0, 1))
    return out_t.reshape(B, TLEN, D), res_g
